# speedup vs baseline: 31.8633x; 31.8633x over previous
import math

import jax
import jax.numpy as jnp
from jax.experimental import pallas as pl
from jax.experimental.pallas import tpu as pltpu

B, Q, S, T = 8, 16, 128, 64
D = 256
H = 8
HD = D // H
STAT_K, TOKEN_K = 8, 16
NEG = -1000000.0
INV_SQRT = 1.0 / math.sqrt(HD)


def _stat_kernel(vl_ref, qf_ref, skf_ref, wqs_ref, wqt_ref, wks_ref, wkt_ref,
                 w_ref, qk_ref):
    qf = qf_ref[...]
    qs = jnp.dot(qf, wqs_ref[...], preferred_element_type=jnp.float32)
    qt = jnp.dot(qf, wqt_ref[...], preferred_element_type=jnp.float32)
    ks = jnp.dot(skf_ref[...], wks_ref[...], preferred_element_type=jnp.float32)

    wkt = wkt_ref[...]
    qk_parts = []
    for h in range(H):
        hsl = slice(h * HD, (h + 1) * HD)
        qk_parts.append(jax.lax.dot_general(
            qt[:, hsl], wkt[:, hsl], (((1,), (1,)), ((), ())),
            preferred_element_type=jnp.float32))
    qk_ref[...] = jnp.stack(qk_parts, axis=0)

    iota_s = jax.lax.broadcasted_iota(jnp.int32, (H * Q, S), 1)
    blocks = []
    for b in range(B):
        qs_b = qs[b * Q:(b + 1) * Q, :]
        ks_b = ks[b * S:(b + 1) * S, :]
        rows_h = []
        for h in range(H):
            hsl = slice(h * HD, (h + 1) * HD)
            rows_h.append(jax.lax.dot_general(
                qs_b[:, hsl], ks_b[:, hsl], (((1,), (1,)), ((), ())),
                preferred_element_type=jnp.float32))
        sc_b = jnp.concatenate(rows_h, axis=0) * INV_SQRT
        vl_b = vl_ref[0, b]
        blocks.append(jnp.where(iota_s < vl_b, sc_b, NEG))
    sc = jnp.concatenate(blocks, axis=0)

    iota_full = jax.lax.broadcasted_iota(jnp.int32, (B * H * Q, S), 1)
    w_dense = jnp.full((B * H * Q, S), NEG, dtype=jnp.float32)
    cur = sc
    big = jnp.int32(2 ** 30)
    for _ in range(STAT_K):
        m = jnp.max(cur, axis=-1, keepdims=True)
        eq = cur == m
        fidx = jnp.min(jnp.where(eq, iota_full, big), axis=-1, keepdims=True)
        oh = iota_full == fidx
        w_dense = jnp.where(oh, cur, w_dense)
        cur = jnp.where(oh, jnp.float32(-3.0e38), cur)
    mx = jnp.max(w_dense, axis=-1, keepdims=True)
    e = jnp.exp(w_dense - mx)
    w_ref[...] = e / jnp.sum(e, axis=-1, keepdims=True)


def _token_kernel(tk_ref, v_ref, qk_ref, w_ref, wv_ref, wo_ref, out_ref):
    tk = tk_ref[...].reshape(S * TOKEN_K, D)
    qk = qk_ref[...].reshape(H * Q, D)
    sc = jax.lax.dot_general(tk, qk, (((1,), (1,)), ((), ())),
                             preferred_element_type=jnp.float32) * INV_SQRT
    sc3 = sc.reshape(S, TOKEN_K, H * Q)
    mx = jnp.max(sc3, axis=1, keepdims=True)
    e = jnp.exp(sc3 - mx)
    a = e / jnp.sum(e, axis=1, keepdims=True)

    wT = w_ref[...].T
    cwf = (a * wT[:, None, :]).reshape(S * TOKEN_K, H * Q)
    m_acc = jax.lax.dot_general(
        cwf, v_ref[...].reshape(S * TOKEN_K, D), (((0,), (0,)), ((), ())),
        preferred_element_type=jnp.float32)

    wv = wv_ref[...]
    parts = []
    for h in range(H):
        hsl = slice(h * HD, (h + 1) * HD)
        parts.append(jnp.dot(m_acc[h * Q:(h + 1) * Q, :], wv[:, hsl],
                             preferred_element_type=jnp.float32))
    out_pre = jnp.concatenate(parts, axis=1)
    out_ref[...] = jnp.dot(out_pre, wo_ref[...],
                           preferred_element_type=jnp.float32)[None]


def kernel(queries, stat_keys, token_keys, values, stat_valid_lens,
           W_q_stat, W_q_token, W_k_stat, W_k_token, W_v, W_o):
    qf = queries.reshape(B * Q, D)
    skf = stat_keys.reshape(B * S, D)
    vl = stat_valid_lens.reshape(1, B).astype(jnp.int32)

    w_flat, qk = pl.pallas_call(
        _stat_kernel,
        out_shape=[
            jax.ShapeDtypeStruct((B * H * Q, S), jnp.float32),
            jax.ShapeDtypeStruct((H, B * Q, D), jnp.float32),
        ],
        in_specs=[
            pl.BlockSpec(memory_space=pltpu.SMEM),
            pl.BlockSpec(memory_space=pltpu.VMEM),
            pl.BlockSpec(memory_space=pltpu.VMEM),
            pl.BlockSpec(memory_space=pltpu.VMEM),
            pl.BlockSpec(memory_space=pltpu.VMEM),
            pl.BlockSpec(memory_space=pltpu.VMEM),
            pl.BlockSpec(memory_space=pltpu.VMEM),
        ],
        out_specs=[
            pl.BlockSpec(memory_space=pltpu.VMEM),
            pl.BlockSpec(memory_space=pltpu.VMEM),
        ],
    )(vl, qf, skf, W_q_stat, W_q_token, W_k_stat, W_k_token)

    t0 = (T - TOKEN_K) // TOKEN_K
    out = pl.pallas_call(
        _token_kernel,
        grid=(B,),
        in_specs=[
            pl.BlockSpec((S, TOKEN_K, D), lambda b: (b, t0, 0)),
            pl.BlockSpec((S, TOKEN_K, D), lambda b: (b, t0, 0)),
            pl.BlockSpec((H, Q, D), lambda b: (0, b, 0)),
            pl.BlockSpec((H * Q, S), lambda b: (b, 0)),
            pl.BlockSpec((D, D), lambda b: (0, 0)),
            pl.BlockSpec((D, D), lambda b: (0, 0)),
        ],
        out_specs=pl.BlockSpec((1, Q, D), lambda b: (b, 0, 0)),
        out_shape=jax.ShapeDtypeStruct((B, Q, D), jnp.float32),
    )(token_keys, values, qk, w_flat, W_v, W_o)
    return out
